# Initial kernel scaffold; baseline (speedup 1.0000x reference)
import jax
import jax.numpy as jnp
from jax.experimental import pallas as pl


def kernel(x, edge_index, W1, b1, W2, b2):
    raise NotImplementedError("write your pallas kernel here")


# speedup vs baseline: 12.5109x; 12.5109x over previous
import functools

import jax
import jax.numpy as jnp
from jax import lax
from jax.experimental import pallas as pl
from jax.experimental.pallas import tpu as pltpu
from jax.experimental.pallas import tpu_sc as plsc

N_NODES_ = 10000
N_PAD = 10240
E_EDGES = 320000
E_PAD = 327680
NC, NS = 2, 16
NW = NC * NS
EPT = E_PAD // NW
ROWS_PT = EPT // 128
STRIPE = N_PAD // NS

_sc_mesh = functools.partial(
    plsc.VectorSubcoreMesh, core_axis_name="c", subcore_axis_name="s")


@functools.partial(
    pl.kernel,
    out_type=jax.ShapeDtypeStruct((NC, N_PAD, 16), jnp.float32),
    mesh=_sc_mesh(),
    compiler_params=pltpu.CompilerParams(use_tc_tiling_on_sc=False),
    scratch_types=[
        pltpu.VMEM((ROWS_PT, 128), jnp.int32),
        pltpu.VMEM((128, 16), jnp.float32),
        pltpu.VMEM((128, 16), jnp.float32),
        pltpu.VMEM_SHARED((N_PAD, 16), jnp.float32),
    ],
)
def _deg_kernel(dst_hbm, out_hbm, didx, ones_v, zeros_v, acc):
    c = lax.axis_index("c")
    s = lax.axis_index("s")
    wid = s * NC + c

    def fill_body(i, _):
        ones_v[i, :] = jnp.ones((16,), jnp.float32)
        zeros_v[i, :] = jnp.zeros((16,), jnp.float32)
        return _

    lax.fori_loop(0, 128, fill_body, None)

    def zcopy_body(r, _):
        pltpu.sync_copy(zeros_v, acc.at[pl.ds(s * STRIPE + r * 128, 128)])
        return _

    lax.fori_loop(0, STRIPE // 128, zcopy_body, None)
    plsc.subcore_barrier()
    pltpu.sync_copy(dst_hbm.at[pl.ds(wid * ROWS_PT, ROWS_PT)], didx)

    def body(b, _):
        pltpu.sync_copy(ones_v, acc.at[didx.at[b]], add=True)
        return _

    lax.fori_loop(0, ROWS_PT, body, None)
    plsc.subcore_barrier()
    pltpu.sync_copy(acc.at[pl.ds(s * STRIPE, STRIPE)],
                    out_hbm.at[c, pl.ds(s * STRIPE, STRIPE)])


def _make_agg_kernel(F):
    @functools.partial(
        pl.kernel,
        out_type=jax.ShapeDtypeStruct((NC, N_PAD, F), jnp.float32),
        mesh=_sc_mesh(),
        compiler_params=pltpu.CompilerParams(use_tc_tiling_on_sc=False),
        scratch_types=[
            pltpu.VMEM((ROWS_PT, 128), jnp.int32),
            pltpu.VMEM((ROWS_PT, 128), jnp.int32),
            pltpu.VMEM((128, F), jnp.float32),
            pltpu.VMEM_SHARED((N_PAD, F), jnp.float32),
            pltpu.SemaphoreType.DMA,
        ],
    )
    def _agg(src_hbm, dst_hbm, p_hbm, out_hbm, sidx, didx, rows, acc, sem):
        c = lax.axis_index("c")
        s = lax.axis_index("s")
        wid = s * NC + c

        nvec = F // 16

        def zero_body(i, _):
            rows[i // nvec, pl.ds((i % nvec) * 16, 16)] = jnp.zeros(
                (16,), jnp.float32)
            return _

        lax.fori_loop(0, 128 * nvec, zero_body, None)

        def zcopy_body(r, _):
            pltpu.sync_copy(rows, acc.at[pl.ds(s * STRIPE + r * 128, 128)])
            return _

        lax.fori_loop(0, STRIPE // 128, zcopy_body, None)
        plsc.subcore_barrier()

        pltpu.sync_copy(src_hbm.at[pl.ds(wid * ROWS_PT, ROWS_PT)], sidx)
        pltpu.sync_copy(dst_hbm.at[pl.ds(wid * ROWS_PT, ROWS_PT)], didx)

        def body(b, _):
            pltpu.async_copy(p_hbm.at[sidx.at[b]], rows, sem).wait()
            pltpu.sync_copy(rows, acc.at[didx.at[b]], add=True)
            return _

        lax.fori_loop(0, ROWS_PT, body, None)
        plsc.subcore_barrier()
        pltpu.sync_copy(acc.at[pl.ds(s * STRIPE, STRIPE)],
                        out_hbm.at[c, pl.ds(s * STRIPE, STRIPE)])

    return _agg


_agg128 = _make_agg_kernel(128)
_agg64 = _make_agg_kernel(64)


_BLK = 1024


def _s_from_deg(degT_ref):
    return lax.rsqrt(
        1.0 + jnp.sum(degT_ref[...], axis=1, keepdims=True) * (1.0 / 16.0))


def _scale_body(degT_ref, x_ref, p_ref):
    p_ref[...] = _s_from_deg(degT_ref) * x_ref[...]


def _mid_body(agg_ref, degT_ref, x_ref, w1_ref, b1_ref, w2_ref, z_ref, p2_ref):
    s = _s_from_deg(degT_ref)
    q = s * (agg_ref[0] + agg_ref[1]) + (s * s) * x_ref[...]
    h = jnp.maximum(
        jnp.dot(q, w1_ref[...], preferred_element_type=jnp.float32)
        + b1_ref[...], 0.0)
    z = jnp.dot(h, w2_ref[...], preferred_element_type=jnp.float32)
    z_ref[...] = z
    p2_ref[...] = s * z


def _final_body(agg_ref, degT_ref, z_ref, b2_ref, out_ref):
    s = _s_from_deg(degT_ref)
    out_ref[...] = s * (agg_ref[0] + agg_ref[1]) + (s * s) * z_ref[...] \
        + b2_ref[...]


def _row_spec(f):
    return pl.BlockSpec((_BLK, f), lambda i: (i, 0))


def _agg_spec(f):
    return pl.BlockSpec((NC, _BLK, f), lambda i: (0, i, 0))


def _full_spec(a, b):
    return pl.BlockSpec((a, b), lambda i: (0, 0))


_GRID = (N_PAD // _BLK,)


def _tc_scale(degT, x_pad):
    return pl.pallas_call(
        _scale_body,
        grid=_GRID,
        in_specs=[_row_spec(NW), _row_spec(128)],
        out_specs=_row_spec(128),
        out_shape=jax.ShapeDtypeStruct((N_PAD, 128), jnp.float32),
    )(degT, x_pad)


def _tc_mid(agg1, degT, x_pad, W1, b1, W2):
    return pl.pallas_call(
        _mid_body,
        grid=_GRID,
        in_specs=[_agg_spec(128), _row_spec(NW), _row_spec(128),
                  _full_spec(128, 256), _full_spec(1, 256),
                  _full_spec(256, 64)],
        out_specs=[_row_spec(64), _row_spec(64)],
        out_shape=[jax.ShapeDtypeStruct((N_PAD, 64), jnp.float32),
                   jax.ShapeDtypeStruct((N_PAD, 64), jnp.float32)],
    )(agg1, degT, x_pad, W1, b1, W2)


def _tc_final(agg2, degT, z, b2):
    return pl.pallas_call(
        _final_body,
        grid=_GRID,
        in_specs=[_agg_spec(64), _row_spec(NW), _row_spec(64),
                  _full_spec(1, 64)],
        out_specs=_row_spec(64),
        out_shape=jax.ShapeDtypeStruct((N_PAD, 64), jnp.float32),
    )(agg2, degT, z, b2)


def kernel(x, edge_index, W1, b1, W2, b2):
    ei = edge_index.astype(jnp.int32)
    pad = jnp.full((E_PAD - E_EDGES,), N_NODES_, dtype=jnp.int32)
    src2d = jnp.concatenate([ei[0], pad]).reshape(E_PAD // 128, 128)
    dst2d = jnp.concatenate([ei[1], pad]).reshape(E_PAD // 128, 128)
    x_pad = jnp.pad(x, ((0, N_PAD - N_NODES_), (0, 0)))

    deg_parts = _deg_kernel(dst2d)
    degT = deg_parts.transpose(1, 0, 2).reshape(N_PAD, NW)

    p1 = _tc_scale(degT, x_pad)
    agg1 = _agg128(src2d, dst2d, p1)
    z, p2 = _tc_mid(agg1, degT, x_pad, W1, b1.reshape(1, 256), W2)
    agg2 = _agg64(src2d, dst2d, p2)
    out = _tc_final(agg2, degT, z, b2.reshape(1, 64))
    return out[:N_NODES_]
